# baseline (speedup 1.0000x reference)
import functools

import jax
import jax.numpy as jnp
from jax import lax
from jax.experimental import pallas as pl
from jax.experimental.pallas import tpu as pltpu
from jax.experimental.pallas import tpu_sc as plsc

N_NODES = 10000
N_GRAPHS = 64
HIDDEN = 512

NCORES = 2
NSUB = 16
NW = 32
BLK = 128
SC_ROWS = NW * BLK
TCB = 256
TC_ROWS = 9984 - SC_ROWS
NTILES = TC_ROWS // TCB
TAIL = 16
TAIL_BASE = 9984
ZROWS = N_GRAPHS // NSUB

_mesh = plsc.VectorSubcoreMesh(core_axis_name="c", subcore_axis_name="s")


@functools.partial(
    pl.kernel,
    out_type=jax.ShapeDtypeStruct((NCORES, N_GRAPHS, 128), jnp.float32),
    mesh=_mesh,
    scratch_types=[
        pltpu.VMEM((BLK, 128), jnp.float32),
        pltpu.VMEM((TAIL, 128), jnp.float32),
        pltpu.VMEM((BLK,), jnp.int32),
        pltpu.VMEM((TAIL,), jnp.int32),
        pltpu.VMEM((ZROWS, 128), jnp.float32),
        pltpu.VMEM_SHARED((N_GRAPHS, 128), jnp.float32),
        pltpu.SemaphoreType.DMA,
        pltpu.SemaphoreType.DMA,
        pltpu.SemaphoreType.DMA,
    ],
)
def _sc_segment_sums(x_hbm, ids_hbm, sums_hbm,
                     xbuf, xtail, idsb, idstail, zsums, shsums,
                     semi, semx, sems):
    cid = lax.axis_index("c")
    sid = lax.axis_index("s")
    w = sid * NCORES + cid
    base = w * BLK
    zero16 = jnp.zeros((16,), jnp.float32)

    load = pltpu.async_copy(x_hbm.at[pl.ds(base, BLK)], xbuf, semx)
    idload = pltpu.async_copy(ids_hbm.at[pl.ds(base, BLK)], idsb, semi)

    @pl.loop(0, ZROWS)
    def _(r):
        @pl.loop(0, 128, step=16)
        def _(c2):
            zsums.at[r, pl.ds(c2, 16)][...] = zero16

    pltpu.sync_copy(zsums, shsums.at[pl.ds(sid * ZROWS, ZROWS)])
    idload.wait()
    plsc.subcore_barrier()

    load.wait()
    scatter = pltpu.async_copy(xbuf, shsums.at[idsb], sems, add=True)

    @pl.when(w == 0)
    def _():
        pltpu.sync_copy(ids_hbm.at[pl.ds(TAIL_BASE, TAIL)], idstail)
        pltpu.sync_copy(x_hbm.at[pl.ds(TAIL_BASE, TAIL)], xtail)
        pltpu.sync_copy(xtail, shsums.at[idstail], add=True)

    scatter.wait()
    plsc.subcore_barrier()

    pltpu.sync_copy(shsums.at[pl.ds(sid * ZROWS, ZROWS)],
                    sums_hbm.at[cid].at[pl.ds(sid * ZROWS, ZROWS)])


def _tc_partial_body(x_ref, ids3_ref, ball_ref, out_ref, cnt_ref):
    i = pl.program_id(0)
    ids = ids3_ref[0, 0]
    seg_iota = lax.broadcasted_iota(jnp.int32, (N_GRAPHS, TCB), 0)
    onehot_t = (ids[None, :] == seg_iota).astype(jnp.float32)
    partial = lax.dot_general(
        onehot_t, x_ref[...], (((1,), (0,)), ((), ())),
        preferred_element_type=jnp.float32)

    @pl.when(i == 0)
    def _():
        all_iota = lax.broadcasted_iota(jnp.int32, (N_GRAPHS, N_NODES), 0)
        onehot_all = (ball_ref[...][None, :] == all_iota).astype(jnp.float32)
        cnt_ref[...] = jnp.sum(onehot_all, axis=1)[:, None]
        out_ref[...] = partial

    @pl.when(i > 0)
    def _():
        out_ref[...] += partial


def _tc_mlp_body(sp_ref, tp_ref, cnt_ref, u_ref, w1u_ref, w1x_ref, b1_ref,
                 w2_ref, b2_ref, w3_ref, b3_ref, out_ref):
    sums = sp_ref[0] + sp_ref[1] + tp_ref[...]
    mean = sums / jnp.maximum(cnt_ref[...], 1.0)
    h = (u_ref[...] @ w1u_ref[...]
         + lax.dot_general(mean, w1x_ref[...], (((1,), (0,)), ((), ())),
                           preferred_element_type=jnp.float32)
         + b1_ref[...])
    h = jnp.maximum(h, 0.0)
    h = lax.dot_general(h, w2_ref[...], (((1,), (0,)), ((), ())),
                        preferred_element_type=jnp.float32) + b2_ref[...]
    h = jnp.maximum(h, 0.0)
    out_ref[...] = lax.dot_general(h, w3_ref[...], (((1,), (0,)), ((), ())),
                                   preferred_element_type=jnp.float32
                                   ) + b3_ref[...]


def kernel(x, edge_index, edge_attr, u, batch, W1, b1, W2, b2, W3, b3):
    del edge_index, edge_attr
    batch32 = batch.astype(jnp.int32)
    sums_p = _sc_segment_sums(x, batch32)
    ids_tc = lax.slice(batch32, (SC_ROWS,), (SC_ROWS + TC_ROWS,))
    ids_tc3 = ids_tc.reshape(NTILES, 1, TCB)
    tc_part, cnt = pl.pallas_call(
        _tc_partial_body,
        grid=(NTILES,),
        in_specs=[
            pl.BlockSpec((TCB, 128), lambda i: (SC_ROWS // TCB + i, 0)),
            pl.BlockSpec((1, 1, TCB), lambda i: (i, 0, 0)),
            pl.BlockSpec((N_NODES,), lambda i: (0,)),
        ],
        out_specs=[
            pl.BlockSpec((N_GRAPHS, 128), lambda i: (0, 0)),
            pl.BlockSpec((N_GRAPHS, 1), lambda i: (0, 0)),
        ],
        out_shape=[
            jax.ShapeDtypeStruct((N_GRAPHS, 128), jnp.float32),
            jax.ShapeDtypeStruct((N_GRAPHS, 1), jnp.float32),
        ],
    )(x, ids_tc3, batch32)
    u2 = u.reshape(N_GRAPHS, 6)
    W1u = W1[:6]
    W1x = W1[6:]
    return pl.pallas_call(
        _tc_mlp_body,
        out_shape=jax.ShapeDtypeStruct((N_GRAPHS, 128), jnp.float32),
    )(sums_p, tc_part, cnt, u2, W1u, W1x, b1.reshape(1, HIDDEN), W2,
      b2.reshape(1, HIDDEN), W3, b3.reshape(1, 128))


# speedup vs baseline: 1.3317x; 1.3317x over previous
import functools

import jax
import jax.numpy as jnp
from jax import lax
from jax.experimental import pallas as pl
from jax.experimental.pallas import tpu as pltpu
from jax.experimental.pallas import tpu_sc as plsc

N_NODES = 10000
N_GRAPHS = 64
HIDDEN = 512

NCORES = 2
NSUB = 16
NW = 32
BLK = 80
NBLK = 2
ROWS_PER_W = BLK * NBLK
SC_BASE = 1536
SC_ROWS = NW * ROWS_PER_W
TCA = SC_BASE
TCB_BASE = SC_BASE + SC_ROWS
TCB = 3328
TCC_BASE = 9984
TCC = 16
ZROWS = N_GRAPHS // NSUB

_mesh = plsc.VectorSubcoreMesh(core_axis_name="c", subcore_axis_name="s")


@functools.partial(
    pl.kernel,
    out_type=jax.ShapeDtypeStruct((NCORES, N_GRAPHS, 128), jnp.float32),
    mesh=_mesh,
    scratch_types=[
        pltpu.VMEM((NBLK, BLK, 128), jnp.float32),
        pltpu.VMEM((NBLK, BLK), jnp.int32),
        pltpu.VMEM((ZROWS, 128), jnp.float32),
        pltpu.VMEM_SHARED((N_GRAPHS, 128), jnp.float32),
        pltpu.SemaphoreType.DMA,
        pltpu.SemaphoreType.DMA,
        pltpu.SemaphoreType.DMA,
        pltpu.SemaphoreType.DMA,
    ],
)
def _sc_segment_sums(x_hbm, ids_hbm, sums_hbm,
                     xbuf, idsb, zsums, shsums, semi, semx0, semx1, sems):
    cid = lax.axis_index("c")
    sid = lax.axis_index("s")
    w = sid * NCORES + cid
    base = SC_BASE + w * ROWS_PER_W
    zero16 = jnp.zeros((16,), jnp.float32)
    semx = (semx0, semx1)

    loads = []
    idloads = []
    for j in range(NBLK):
        loads.append(pltpu.async_copy(
            x_hbm.at[pl.ds(base + j * BLK, BLK)], xbuf.at[j], semx[j]))
        idloads.append(pltpu.async_copy(
            ids_hbm.at[pl.ds(base + j * BLK, BLK)], idsb.at[j], semi))

    @pl.loop(0, ZROWS)
    def _(r):
        @pl.loop(0, 128, step=16)
        def _(c2):
            zsums.at[r, pl.ds(c2, 16)][...] = zero16

    pltpu.sync_copy(zsums, shsums.at[pl.ds(sid * ZROWS, ZROWS)])
    for j in range(NBLK):
        idloads[j].wait()
    plsc.subcore_barrier()

    scatters = []
    for j in range(NBLK):
        loads[j].wait()
        scatters.append(pltpu.async_copy(
            xbuf.at[j], shsums.at[idsb.at[j]], sems, add=True))
    for j in range(NBLK):
        scatters[j].wait()
    plsc.subcore_barrier()

    pltpu.sync_copy(shsums.at[pl.ds(sid * ZROWS, ZROWS)],
                    sums_hbm.at[cid].at[pl.ds(sid * ZROWS, ZROWS)])


def _onehot_matmul(ids, x_ref, n):
    seg_iota = lax.broadcasted_iota(jnp.int32, (N_GRAPHS, n), 0)
    onehot_t = (ids[None, :] == seg_iota).astype(jnp.bfloat16)
    return lax.dot_general(
        onehot_t, x_ref[...].astype(jnp.bfloat16), (((1,), (0,)), ((), ())),
        preferred_element_type=jnp.float32)


def _tc_partial_body(xa_ref, xb_ref, xc_ref, ia_ref, ib_ref, ic_ref,
                     ball_ref, out_ref, cnt_ref):
    out_ref[...] = (_onehot_matmul(ia_ref[...], xa_ref, TCA)
                    + _onehot_matmul(ib_ref[...], xb_ref, TCB)
                    + _onehot_matmul(ic_ref[...], xc_ref, TCC))
    all_iota = lax.broadcasted_iota(jnp.int32, (N_GRAPHS, N_NODES), 0)
    onehot_all = (ball_ref[...][None, :] == all_iota).astype(jnp.float32)
    cnt_ref[...] = jnp.sum(onehot_all, axis=1)[:, None]


def _tc_mlp_body(sp_ref, tp_ref, cnt_ref, u_ref, w1u_ref, w1x_ref, b1_ref,
                 w2_ref, b2_ref, w3_ref, b3_ref, out_ref):
    sums = sp_ref[0] + sp_ref[1] + tp_ref[...]
    mean = (sums / jnp.maximum(cnt_ref[...], 1.0)).astype(jnp.bfloat16)
    h = (lax.dot_general(u_ref[...], w1u_ref[...], (((1,), (0,)), ((), ())),
                         preferred_element_type=jnp.float32)
         + lax.dot_general(mean, w1x_ref[...], (((1,), (0,)), ((), ())),
                           preferred_element_type=jnp.float32)
         + b1_ref[...])
    h = jnp.maximum(h, 0.0).astype(jnp.bfloat16)
    h = lax.dot_general(h, w2_ref[...], (((1,), (0,)), ((), ())),
                        preferred_element_type=jnp.float32) + b2_ref[...]
    h = jnp.maximum(h, 0.0).astype(jnp.bfloat16)
    out_ref[...] = lax.dot_general(h, w3_ref[...], (((1,), (0,)), ((), ())),
                                   preferred_element_type=jnp.float32
                                   ) + b3_ref[...]


def kernel(x, edge_index, edge_attr, u, batch, W1, b1, W2, b2, W3, b3):
    del edge_index, edge_attr
    batch32 = batch.astype(jnp.int32)
    sums_p = _sc_segment_sums(x, batch32)
    ids_a = lax.slice(batch32, (0,), (TCA,))
    ids_b = lax.slice(batch32, (TCB_BASE,), (TCB_BASE + TCB,))
    ids_c = lax.slice(batch32, (TCC_BASE,), (TCC_BASE + TCC,))
    tc_part, cnt = pl.pallas_call(
        _tc_partial_body,
        grid=(1,),
        in_specs=[
            pl.BlockSpec((TCA, 128), lambda i: (0, 0)),
            pl.BlockSpec((TCB, 128), lambda i: (TCB_BASE // TCB, 0)),
            pl.BlockSpec((TCC, 128), lambda i: (TCC_BASE // TCC, 0)),
            pl.BlockSpec((TCA,), lambda i: (0,)),
            pl.BlockSpec((TCB,), lambda i: (0,)),
            pl.BlockSpec((TCC,), lambda i: (0,)),
            pl.BlockSpec((N_NODES,), lambda i: (0,)),
        ],
        out_specs=[
            pl.BlockSpec((N_GRAPHS, 128), lambda i: (0, 0)),
            pl.BlockSpec((N_GRAPHS, 1), lambda i: (0, 0)),
        ],
        out_shape=[
            jax.ShapeDtypeStruct((N_GRAPHS, 128), jnp.float32),
            jax.ShapeDtypeStruct((N_GRAPHS, 1), jnp.float32),
        ],
    )(x, x, x, ids_a, ids_b, ids_c, batch32)
    u2 = u.reshape(N_GRAPHS, 6).astype(jnp.bfloat16)
    W1u = W1[:6].astype(jnp.bfloat16)
    W1x = W1[6:].astype(jnp.bfloat16)
    return pl.pallas_call(
        _tc_mlp_body,
        out_shape=jax.ShapeDtypeStruct((N_GRAPHS, 128), jnp.float32),
    )(sums_p, tc_part, cnt, u2, W1u, W1x, b1.reshape(1, HIDDEN),
      W2.astype(jnp.bfloat16), b2.reshape(1, HIDDEN),
      W3.astype(jnp.bfloat16), b3.reshape(1, 128))


# speedup vs baseline: 1.4522x; 1.0905x over previous
import functools

import jax
import jax.numpy as jnp
from jax import lax
from jax.experimental import pallas as pl
from jax.experimental.pallas import tpu as pltpu
from jax.experimental.pallas import tpu_sc as plsc

N_NODES = 10000
N_GRAPHS = 64
HIDDEN = 512

NCORES = 2
NSUB = 16
NW = 32
BLK = 80
NBLK = 2
ROWS_PER_W = BLK * NBLK
SC_BASE = 1536
SC_ROWS = NW * ROWS_PER_W
TCA = SC_BASE
TCB_BASE = SC_BASE + SC_ROWS
TCB = 3328
TCC_BASE = 9984
TCC = 16
ZROWS = N_GRAPHS // NSUB

_mesh = plsc.VectorSubcoreMesh(core_axis_name="c", subcore_axis_name="s")


@functools.partial(
    pl.kernel,
    out_type=jax.ShapeDtypeStruct((NCORES, N_GRAPHS, 128), jnp.float32),
    mesh=_mesh,
    scratch_types=[
        pltpu.VMEM((NBLK, BLK, 128), jnp.float32),
        pltpu.VMEM((NBLK, BLK), jnp.int32),
        pltpu.VMEM((ZROWS, 128), jnp.float32),
        pltpu.VMEM_SHARED((N_GRAPHS, 128), jnp.float32),
        pltpu.SemaphoreType.DMA,
        pltpu.SemaphoreType.DMA,
        pltpu.SemaphoreType.DMA,
        pltpu.SemaphoreType.DMA,
    ],
)
def _sc_segment_sums(x_hbm, ids_hbm, sums_hbm,
                     xbuf, idsb, zsums, shsums, semi, semx0, semx1, sems):
    cid = lax.axis_index("c")
    sid = lax.axis_index("s")
    w = sid * NCORES + cid
    base = SC_BASE + w * ROWS_PER_W
    zero16 = jnp.zeros((16,), jnp.float32)
    semx = (semx0, semx1)

    loads = []
    idloads = []
    for j in range(NBLK):
        loads.append(pltpu.async_copy(
            x_hbm.at[pl.ds(base + j * BLK, BLK)], xbuf.at[j], semx[j]))
        idloads.append(pltpu.async_copy(
            ids_hbm.at[pl.ds(base + j * BLK, BLK)], idsb.at[j], semi))

    @pl.loop(0, ZROWS)
    def _(r):
        @pl.loop(0, 128, step=16)
        def _(c2):
            zsums.at[r, pl.ds(c2, 16)][...] = zero16

    pltpu.sync_copy(zsums, shsums.at[pl.ds(sid * ZROWS, ZROWS)])
    for j in range(NBLK):
        idloads[j].wait()
    plsc.subcore_barrier()

    scatters = []
    for j in range(NBLK):
        loads[j].wait()
        scatters.append(pltpu.async_copy(
            xbuf.at[j], shsums.at[idsb.at[j]], sems, add=True))
    for j in range(NBLK):
        scatters[j].wait()
    plsc.subcore_barrier()

    pltpu.sync_copy(shsums.at[pl.ds(sid * ZROWS, ZROWS)],
                    sums_hbm.at[cid].at[pl.ds(sid * ZROWS, ZROWS)])


def _onehot_matmul(ids, x_ref, n):
    seg_iota = lax.broadcasted_iota(jnp.int32, (N_GRAPHS, n), 0)
    onehot_t = (ids[None, :] == seg_iota).astype(jnp.bfloat16)
    return lax.dot_general(
        onehot_t, x_ref[...].astype(jnp.bfloat16), (((1,), (0,)), ((), ())),
        preferred_element_type=jnp.float32)


def _tc_partial_body(xa_ref, xb_ref, xc_ref, ball_ref, out_ref, cnt_ref):
    out_ref[...] = (
        _onehot_matmul(ball_ref[pl.ds(0, TCA)], xa_ref, TCA)
        + _onehot_matmul(ball_ref[pl.ds(TCB_BASE, TCB)], xb_ref, TCB)
        + _onehot_matmul(ball_ref[pl.ds(TCC_BASE, TCC)], xc_ref, TCC))
    all_iota = lax.broadcasted_iota(jnp.int32, (N_GRAPHS, N_NODES), 0)
    onehot_all = (ball_ref[...][None, :] == all_iota).astype(jnp.float32)
    cnt_ref[...] = jnp.sum(onehot_all, axis=1)[:, None]


def _tc_mlp_body(sp_ref, tp_ref, cnt_ref, u_ref, w1_ref, b1_ref,
                 w2_ref, b2_ref, w3_ref, b3_ref, out_ref):
    sums = sp_ref[0] + sp_ref[1] + tp_ref[...]
    mean = (sums / jnp.maximum(cnt_ref[...], 1.0)).astype(jnp.bfloat16)
    w1u = w1_ref[0:6, :].astype(jnp.bfloat16)
    w1x = w1_ref[6:, :].astype(jnp.bfloat16)
    h = (lax.dot_general(u_ref[...].astype(jnp.bfloat16), w1u,
                         (((1,), (0,)), ((), ())),
                         preferred_element_type=jnp.float32)
         + lax.dot_general(mean, w1x, (((1,), (0,)), ((), ())),
                           preferred_element_type=jnp.float32)
         + b1_ref[...])
    h = jnp.maximum(h, 0.0).astype(jnp.bfloat16)
    h = lax.dot_general(h, w2_ref[...].astype(jnp.bfloat16),
                        (((1,), (0,)), ((), ())),
                        preferred_element_type=jnp.float32) + b2_ref[...]
    h = jnp.maximum(h, 0.0).astype(jnp.bfloat16)
    out_ref[...] = lax.dot_general(h, w3_ref[...].astype(jnp.bfloat16),
                                   (((1,), (0,)), ((), ())),
                                   preferred_element_type=jnp.float32
                                   ) + b3_ref[...]


def kernel(x, edge_index, edge_attr, u, batch, W1, b1, W2, b2, W3, b3):
    del edge_index, edge_attr
    batch32 = batch.astype(jnp.int32)
    sums_p = _sc_segment_sums(x, batch32)
    tc_part, cnt = pl.pallas_call(
        _tc_partial_body,
        grid=(1,),
        in_specs=[
            pl.BlockSpec((TCA, 128), lambda i: (0, 0)),
            pl.BlockSpec((TCB, 128), lambda i: (TCB_BASE // TCB, 0)),
            pl.BlockSpec((TCC, 128), lambda i: (TCC_BASE // TCC, 0)),
            pl.BlockSpec((N_NODES,), lambda i: (0,)),
        ],
        out_specs=[
            pl.BlockSpec((N_GRAPHS, 128), lambda i: (0, 0)),
            pl.BlockSpec((N_GRAPHS, 1), lambda i: (0, 0)),
        ],
        out_shape=[
            jax.ShapeDtypeStruct((N_GRAPHS, 128), jnp.float32),
            jax.ShapeDtypeStruct((N_GRAPHS, 1), jnp.float32),
        ],
    )(x, x, x, batch32)
    return pl.pallas_call(
        _tc_mlp_body,
        out_shape=jax.ShapeDtypeStruct((N_GRAPHS, 128), jnp.float32),
    )(sums_p, tc_part, cnt, u.reshape(N_GRAPHS, 6), W1,
      b1.reshape(1, HIDDEN), W2, b2.reshape(1, HIDDEN), W3,
      b3.reshape(1, 128))


# speedup vs baseline: 1.4763x; 1.0166x over previous
import functools

import jax
import jax.numpy as jnp
from jax import lax
from jax.experimental import pallas as pl
from jax.experimental.pallas import tpu as pltpu
from jax.experimental.pallas import tpu_sc as plsc

N_NODES = 10000
N_GRAPHS = 64
HIDDEN = 512

NCORES = 2
NSUB = 16
NW = 32
BLK = 80
NBLK = 1
ROWS_PER_W = BLK * NBLK
SC_BASE = 4096
SC_ROWS = NW * ROWS_PER_W
TCA = SC_BASE
TCB_BASE = SC_BASE + SC_ROWS
TCB = 3328
TCC_BASE = 9984
TCC = 16
ZROWS = N_GRAPHS // NSUB

_mesh = plsc.VectorSubcoreMesh(core_axis_name="c", subcore_axis_name="s")


@functools.partial(
    pl.kernel,
    out_type=jax.ShapeDtypeStruct((NCORES, N_GRAPHS, 128), jnp.float32),
    mesh=_mesh,
    scratch_types=[
        pltpu.VMEM((NBLK, BLK, 128), jnp.float32),
        pltpu.VMEM((NBLK, BLK), jnp.int32),
        pltpu.VMEM((ZROWS, 128), jnp.float32),
        pltpu.VMEM_SHARED((N_GRAPHS, 128), jnp.float32),
        pltpu.SemaphoreType.DMA,
        pltpu.SemaphoreType.DMA,
        pltpu.SemaphoreType.DMA,
    ],
)
def _sc_segment_sums(x_hbm, ids_hbm, sums_hbm,
                     xbuf, idsb, zsums, shsums, semi, semx0, sems):
    cid = lax.axis_index("c")
    sid = lax.axis_index("s")
    w = sid * NCORES + cid
    base = SC_BASE + w * ROWS_PER_W
    zero16 = jnp.zeros((16,), jnp.float32)
    semx = (semx0,)

    loads = []
    idloads = []
    for j in range(NBLK):
        loads.append(pltpu.async_copy(
            x_hbm.at[pl.ds(base + j * BLK, BLK)], xbuf.at[j], semx[j]))
        idloads.append(pltpu.async_copy(
            ids_hbm.at[pl.ds(base + j * BLK, BLK)], idsb.at[j], semi))

    @pl.loop(0, ZROWS)
    def _(r):
        @pl.loop(0, 128, step=16)
        def _(c2):
            zsums.at[r, pl.ds(c2, 16)][...] = zero16

    pltpu.sync_copy(zsums, shsums.at[pl.ds(sid * ZROWS, ZROWS)])
    for j in range(NBLK):
        idloads[j].wait()
    plsc.subcore_barrier()

    scatters = []
    for j in range(NBLK):
        loads[j].wait()
        scatters.append(pltpu.async_copy(
            xbuf.at[j], shsums.at[idsb.at[j]], sems, add=True))
    for j in range(NBLK):
        scatters[j].wait()
    plsc.subcore_barrier()

    pltpu.sync_copy(shsums.at[pl.ds(sid * ZROWS, ZROWS)],
                    sums_hbm.at[cid].at[pl.ds(sid * ZROWS, ZROWS)])


def _onehot_matmul(ids, x_ref, n):
    seg_iota = lax.broadcasted_iota(jnp.int32, (N_GRAPHS, n), 0)
    onehot_t = (ids[None, :] == seg_iota).astype(jnp.bfloat16)
    return lax.dot_general(
        onehot_t, x_ref[...].astype(jnp.bfloat16), (((1,), (0,)), ((), ())),
        preferred_element_type=jnp.float32)


def _tc_partial_body(xa_ref, xb_ref, xc_ref, ball_ref, out_ref, cnt_ref):
    out_ref[...] = (
        _onehot_matmul(ball_ref[pl.ds(0, TCA)], xa_ref, TCA)
        + _onehot_matmul(ball_ref[pl.ds(TCB_BASE, TCB)], xb_ref, TCB)
        + _onehot_matmul(ball_ref[pl.ds(TCC_BASE, TCC)], xc_ref, TCC))
    all_iota = lax.broadcasted_iota(jnp.int32, (N_GRAPHS, N_NODES), 0)
    onehot_all = (ball_ref[...][None, :] == all_iota).astype(jnp.float32)
    cnt_ref[...] = jnp.sum(onehot_all, axis=1)[:, None]


def _tc_mlp_body(sp_ref, tp_ref, cnt_ref, u_ref, w1_ref, b1_ref,
                 w2_ref, b2_ref, w3_ref, b3_ref, out_ref):
    sums = sp_ref[0] + sp_ref[1] + tp_ref[...]
    mean = (sums / jnp.maximum(cnt_ref[...], 1.0)).astype(jnp.bfloat16)
    w1u = w1_ref[0:6, :].astype(jnp.bfloat16)
    w1x = w1_ref[6:, :].astype(jnp.bfloat16)
    h = (lax.dot_general(u_ref[...].astype(jnp.bfloat16), w1u,
                         (((1,), (0,)), ((), ())),
                         preferred_element_type=jnp.float32)
         + lax.dot_general(mean, w1x, (((1,), (0,)), ((), ())),
                           preferred_element_type=jnp.float32)
         + b1_ref[...])
    h = jnp.maximum(h, 0.0).astype(jnp.bfloat16)
    h = lax.dot_general(h, w2_ref[...].astype(jnp.bfloat16),
                        (((1,), (0,)), ((), ())),
                        preferred_element_type=jnp.float32) + b2_ref[...]
    h = jnp.maximum(h, 0.0).astype(jnp.bfloat16)
    out_ref[...] = lax.dot_general(h, w3_ref[...].astype(jnp.bfloat16),
                                   (((1,), (0,)), ((), ())),
                                   preferred_element_type=jnp.float32
                                   ) + b3_ref[...]


def kernel(x, edge_index, edge_attr, u, batch, W1, b1, W2, b2, W3, b3):
    del edge_index, edge_attr
    batch32 = batch.astype(jnp.int32)
    sums_p = _sc_segment_sums(x, batch32)
    tc_part, cnt = pl.pallas_call(
        _tc_partial_body,
        grid=(1,),
        in_specs=[
            pl.BlockSpec((TCA, 128), lambda i: (0, 0)),
            pl.BlockSpec((TCB, 128), lambda i: (TCB_BASE // TCB, 0)),
            pl.BlockSpec((TCC, 128), lambda i: (TCC_BASE // TCC, 0)),
            pl.BlockSpec((N_NODES,), lambda i: (0,)),
        ],
        out_specs=[
            pl.BlockSpec((N_GRAPHS, 128), lambda i: (0, 0)),
            pl.BlockSpec((N_GRAPHS, 1), lambda i: (0, 0)),
        ],
        out_shape=[
            jax.ShapeDtypeStruct((N_GRAPHS, 128), jnp.float32),
            jax.ShapeDtypeStruct((N_GRAPHS, 1), jnp.float32),
        ],
    )(x, x, x, batch32)
    return pl.pallas_call(
        _tc_mlp_body,
        out_shape=jax.ShapeDtypeStruct((N_GRAPHS, 128), jnp.float32),
    )(sums_p, tc_part, cnt, u.reshape(N_GRAPHS, 6), W1,
      b1.reshape(1, HIDDEN), W2, b2.reshape(1, HIDDEN), W3,
      b3.reshape(1, 128))


# speedup vs baseline: 1.4857x; 1.0063x over previous
import functools

import jax
import jax.numpy as jnp
from jax import lax
from jax.experimental import pallas as pl
from jax.experimental.pallas import tpu as pltpu
from jax.experimental.pallas import tpu_sc as plsc

N_NODES = 10000
N_GRAPHS = 64
HIDDEN = 512

NCORES = 2
NSUB = 16
NW = 32
BLK = 64
NBLK = 1
ROWS_PER_W = BLK * NBLK
SC_BASE = 4608
SC_ROWS = NW * ROWS_PER_W
TCA = SC_BASE
TCB_BASE = SC_BASE + SC_ROWS
TCB = 3328
TCC_BASE = 9984
TCC = 16
ZROWS = N_GRAPHS // NSUB

_mesh = plsc.VectorSubcoreMesh(core_axis_name="c", subcore_axis_name="s")


@functools.partial(
    pl.kernel,
    out_type=jax.ShapeDtypeStruct((NCORES, N_GRAPHS, 128), jnp.float32),
    mesh=_mesh,
    scratch_types=[
        pltpu.VMEM((NBLK, BLK, 128), jnp.float32),
        pltpu.VMEM((NBLK, BLK), jnp.int32),
        pltpu.VMEM((ZROWS, 128), jnp.float32),
        pltpu.VMEM_SHARED((N_GRAPHS, 128), jnp.float32),
        pltpu.SemaphoreType.DMA,
        pltpu.SemaphoreType.DMA,
        pltpu.SemaphoreType.DMA,
    ],
)
def _sc_segment_sums(x_hbm, ids_hbm, sums_hbm,
                     xbuf, idsb, zsums, shsums, semi, semx0, sems):
    cid = lax.axis_index("c")
    sid = lax.axis_index("s")
    w = sid * NCORES + cid
    base = SC_BASE + w * ROWS_PER_W
    zero16 = jnp.zeros((16,), jnp.float32)
    semx = (semx0,)

    loads = []
    idloads = []
    for j in range(NBLK):
        loads.append(pltpu.async_copy(
            x_hbm.at[pl.ds(base + j * BLK, BLK)], xbuf.at[j], semx[j]))
        idloads.append(pltpu.async_copy(
            ids_hbm.at[pl.ds(base + j * BLK, BLK)], idsb.at[j], semi))

    @pl.loop(0, ZROWS)
    def _(r):
        @pl.loop(0, 128, step=16)
        def _(c2):
            zsums.at[r, pl.ds(c2, 16)][...] = zero16

    pltpu.sync_copy(zsums, shsums.at[pl.ds(sid * ZROWS, ZROWS)])
    for j in range(NBLK):
        idloads[j].wait()
    plsc.subcore_barrier()

    scatters = []
    for j in range(NBLK):
        loads[j].wait()
        scatters.append(pltpu.async_copy(
            xbuf.at[j], shsums.at[idsb.at[j]], sems, add=True))
    for j in range(NBLK):
        scatters[j].wait()
    plsc.subcore_barrier()

    pltpu.sync_copy(shsums.at[pl.ds(sid * ZROWS, ZROWS)],
                    sums_hbm.at[cid].at[pl.ds(sid * ZROWS, ZROWS)])


def _onehot_matmul(ids, x_ref, n):
    seg_iota = lax.broadcasted_iota(jnp.int32, (N_GRAPHS, n), 0)
    onehot_t = (ids[None, :] == seg_iota).astype(jnp.bfloat16)
    return lax.dot_general(
        onehot_t, x_ref[...].astype(jnp.bfloat16), (((1,), (0,)), ((), ())),
        preferred_element_type=jnp.float32)


def _tc_partial_body(xa_ref, xb_ref, xc_ref, ball_ref, out_ref, cnt_ref):
    out_ref[...] = (
        _onehot_matmul(ball_ref[pl.ds(0, TCA)], xa_ref, TCA)
        + _onehot_matmul(ball_ref[pl.ds(TCB_BASE, TCB)], xb_ref, TCB)
        + _onehot_matmul(ball_ref[pl.ds(TCC_BASE, TCC)], xc_ref, TCC))
    all_iota = lax.broadcasted_iota(jnp.int32, (N_GRAPHS, N_NODES), 0)
    onehot_all = (ball_ref[...][None, :] == all_iota).astype(jnp.float32)
    cnt_ref[...] = jnp.sum(onehot_all, axis=1)[:, None]


def _tc_mlp_body(sp_ref, tp_ref, cnt_ref, u_ref, w1_ref, b1_ref,
                 w2_ref, b2_ref, w3_ref, b3_ref, out_ref):
    sums = sp_ref[0] + sp_ref[1] + tp_ref[...]
    mean = (sums / jnp.maximum(cnt_ref[...], 1.0)).astype(jnp.bfloat16)
    w1u = w1_ref[0:6, :].astype(jnp.bfloat16)
    w1x = w1_ref[6:, :].astype(jnp.bfloat16)
    h = (lax.dot_general(u_ref[...].astype(jnp.bfloat16), w1u,
                         (((1,), (0,)), ((), ())),
                         preferred_element_type=jnp.float32)
         + lax.dot_general(mean, w1x, (((1,), (0,)), ((), ())),
                           preferred_element_type=jnp.float32)
         + b1_ref[...])
    h = jnp.maximum(h, 0.0).astype(jnp.bfloat16)
    h = lax.dot_general(h, w2_ref[...].astype(jnp.bfloat16),
                        (((1,), (0,)), ((), ())),
                        preferred_element_type=jnp.float32) + b2_ref[...]
    h = jnp.maximum(h, 0.0).astype(jnp.bfloat16)
    out_ref[...] = lax.dot_general(h, w3_ref[...].astype(jnp.bfloat16),
                                   (((1,), (0,)), ((), ())),
                                   preferred_element_type=jnp.float32
                                   ) + b3_ref[...]


def kernel(x, edge_index, edge_attr, u, batch, W1, b1, W2, b2, W3, b3):
    del edge_index, edge_attr
    batch32 = batch.astype(jnp.int32)
    sums_p = _sc_segment_sums(x, batch32)
    tc_part, cnt = pl.pallas_call(
        _tc_partial_body,
        grid=(1,),
        in_specs=[
            pl.BlockSpec((TCA, 128), lambda i: (0, 0)),
            pl.BlockSpec((TCB, 128), lambda i: (TCB_BASE // TCB, 0)),
            pl.BlockSpec((TCC, 128), lambda i: (TCC_BASE // TCC, 0)),
            pl.BlockSpec((N_NODES,), lambda i: (0,)),
        ],
        out_specs=[
            pl.BlockSpec((N_GRAPHS, 128), lambda i: (0, 0)),
            pl.BlockSpec((N_GRAPHS, 1), lambda i: (0, 0)),
        ],
        out_shape=[
            jax.ShapeDtypeStruct((N_GRAPHS, 128), jnp.float32),
            jax.ShapeDtypeStruct((N_GRAPHS, 1), jnp.float32),
        ],
    )(x, x, x, batch32)
    return pl.pallas_call(
        _tc_mlp_body,
        out_shape=jax.ShapeDtypeStruct((N_GRAPHS, 128), jnp.float32),
    )(sums_p, tc_part, cnt, u.reshape(N_GRAPHS, 6), W1,
      b1.reshape(1, HIDDEN), W2, b2.reshape(1, HIDDEN), W3,
      b3.reshape(1, 128))
